# baseline (speedup 1.0000x reference)
import functools

import jax
import jax.numpy as jnp
import numpy as np
from jax import lax
from jax.experimental import pallas as pl
from jax.experimental.pallas import tpu as pltpu
from jax.experimental.pallas import tpu_sc as plsc

H = 2
R = 3
LANES = 16
NC = 2
NS = 16


def _ln(x, g, b, eps=1e-5):
    m = x.mean(-1, keepdims=True)
    v = ((x - m) ** 2).mean(-1, keepdims=True)
    return (x - m) / jnp.sqrt(v + eps) * g + b


def _popcount32(bm):
    x = bm - ((bm >> 1) & 0x55555555)
    x = (x & 0x33333333) + ((x >> 2) & 0x33333333)
    x = (x + (x >> 4)) & 0x0F0F0F0F
    return (x + (x >> 8) + (x >> 16) + (x >> 24)) & 0x3F


def _ctz(bm):
    low = bm & (-bm)
    return (((lax.bitcast_convert_type(low.astype(jnp.float32), jnp.int32)
              >> 23) & 0xFF) - 127)


def _tconv_edge_sc(q, k, v, src, dst):
    n, W = q.shape
    EP = src.shape[0]
    C = W // H
    NW = NC * NS
    WIN = 256
    RN = n // (NW * WIN)
    S = 4096
    K = 32
    NSEC = EP // S
    NV = S // LANES
    ACC = WIN * W
    ACCD = WIN * LANES
    rscale = 1.0 / np.sqrt(C)

    mesh = plsc.VectorSubcoreMesh(core_axis_name="c", subcore_axis_name="s")

    @functools.partial(
        pl.kernel,
        out_type=[jax.ShapeDtypeStruct((n * W,), jnp.float32),
                  jax.ShapeDtypeStruct((n * LANES,), jnp.float32)],
        mesh=mesh,
        scratch_types=[
            pltpu.VMEM((ACC,), jnp.float32),
            pltpu.VMEM((ACCD,), jnp.float32),
            pltpu.VMEM((S,), jnp.int32),
            pltpu.VMEM((S,), jnp.int32),
            pltpu.VMEM((S + LANES,), jnp.int32),
            pltpu.VMEM((S + LANES,), jnp.int32),
            pltpu.VMEM((K, W), jnp.float32),
            pltpu.VMEM((K, W), jnp.float32),
            pltpu.VMEM((K, W), jnp.float32),
            pltpu.VMEM((K,), jnp.int32),
            pltpu.VMEM((K,), jnp.int32),
            pltpu.SemaphoreType.DMA,
        ],
    )
    def body(q_hbm, k_hbm, v_hbm, src_hbm, dst_hbm, zeros_hbm,
             num_out, den_out,
             acc, accden, sbuf, dbuf, slist, dlist,
             qrows, krows, vrows, sidx, gidx, gsem):
        ci = lax.axis_index("c")
        si = lax.axis_index("s")
        wid = ci * NS + si
        iota = lax.iota(jnp.int32, LANES)
        onev = jnp.int32(1) << iota
        onevh = onev << LANES
        zv = jnp.zeros((LANES,), jnp.float32)
        ziv = jnp.zeros((LANES,), jnp.int32)
        d1 = iota ^ 1
        f0v = ((((iota | -iota) >> 31) & 1) ^ 1).astype(jnp.float32)
        f1v = ((((d1 | -d1) >> 31) & 1) ^ 1).astype(jnp.float32)

        def round_body(r, _0):
            wdw = wid * RN + r
            nbase = wdw * WIN
            pltpu.sync_copy(zeros_hbm.at[pl.ds(0, ACC)], acc)
            pltpu.sync_copy(zeros_hbm.at[pl.ds(0, ACCD)], accden)

            def sec_body(sec, _1):
                pltpu.sync_copy(src_hbm.at[pl.ds(sec * S, S)], sbuf)
                pltpu.sync_copy(dst_hbm.at[pl.ds(sec * S, S)], dbuf)

                def scan_body(i, carry):
                    cnt, stS, stD = carry
                    d0 = dbuf[pl.ds(i * 32, LANES)]
                    d1 = dbuf[pl.ds(i * 32 + LANES, LANES)]
                    dl0 = d0 - nbase
                    dl1 = d1 - nbase
                    neg0 = (dl0 | (WIN - 1 - dl0)) >> 31
                    neg1 = (dl1 | (WIN - 1 - dl1)) >> 31
                    bits = (onev & ~neg0) | (onevh & ~neg1)
                    for st_ in (8, 4, 2, 1):
                        bits = bits | bits[iota ^ st_]
                    bm0 = bits[0]
                    npop = _popcount32(bm0)
                    sv0 = sbuf[pl.ds(i * 32, LANES)]
                    sv1 = sbuf[pl.ds(i * 32 + LANES, LANES)]

                    def ext_body(j, c2):
                        bm, cnt2, sS, sD = c2
                        lane = _ctz(bm)
                        lsp = jnp.full((LANES,), lane & (LANES - 1))
                        him = jnp.full((LANES,), -(lane >> 4))
                        sval = (sv0[lsp] & ~him) | (sv1[lsp] & him)
                        dval = (dl0[lsp] & ~him) | (dl1[lsp] & him)
                        dslot = iota ^ (cnt2 & (LANES - 1))
                        keepm = (dslot | -dslot) >> 31
                        sS = (sS & keepm) | (sval & ~keepm)
                        sD = (sD & keepm) | (dval & ~keepm)

                        @pl.when((cnt2 & (LANES - 1)) == (LANES - 1))
                        def _():
                            slist[pl.ds(cnt2 - (LANES - 1), LANES)] = sS
                            dlist[pl.ds(cnt2 - (LANES - 1), LANES)] = sD

                        return (bm & (bm - 1), cnt2 + 1, sS, sD)

                    bm, cnt, stS, stD = lax.fori_loop(
                        0, npop, ext_body, (bm0, cnt, stS, stD))
                    return (cnt, stS, stD)

                cnt, stS, stD = lax.fori_loop(
                    0, NV // 2, scan_body, (jnp.int32(0), ziv, ziv))

                @pl.when((cnt & (LANES - 1)) != 0)
                def _():
                    fb = (cnt >> 4) << 4
                    slist[pl.ds(fb, LANES)] = stS
                    dlist[pl.ds(fb, LANES)] = stD

                cs = jnp.full((LANES,), cnt)
                nb = (cnt + K - 1) // K

                def drain_body(b, _2):
                    off = b * K
                    for t_ in range(K // LANES):
                        validm = ((iota + off + t_ * LANES) - cs) >> 31
                        sv = slist[pl.ds(off + t_ * LANES, LANES)] & validm
                        dv = dlist[pl.ds(off + t_ * LANES, LANES)] & validm
                        sidx[pl.ds(t_ * LANES, LANES)] = sv
                        gidx[pl.ds(t_ * LANES, LANES)] = dv + (nbase & validm)
                    cq = pltpu.async_copy(q_hbm.at[gidx], qrows, gsem)
                    ck = pltpu.async_copy(k_hbm.at[sidx], krows, gsem)
                    cv = pltpu.async_copy(v_hbm.at[sidx], vrows, gsem)
                    cq.wait()
                    ck.wait()
                    cv.wait()

                    def edge_body(e, _3):
                        okf = jnp.full(
                            (LANES,),
                            ((((off + e) - cnt) >> 31) & 1).astype(jnp.float32))
                        a0 = qrows[e, pl.ds(0, LANES)] * krows[e, pl.ds(0, LANES)]
                        a1 = qrows[e, pl.ds(C, LANES)] * krows[e, pl.ds(C, LANES)]
                        for t in range(1, C // LANES):
                            a0 += (qrows[e, pl.ds(t * LANES, LANES)] *
                                   krows[e, pl.ds(t * LANES, LANES)])
                            a1 += (qrows[e, pl.ds(C + t * LANES, LANES)] *
                                   krows[e, pl.ds(C + t * LANES, LANES)])
                        for st_ in (8, 4, 2, 1):
                            a0 = a0 + a0[iota ^ st_]
                            a1 = a1 + a1[iota ^ st_]
                        p0 = jnp.exp(a0 * rscale) * okf
                        p1 = jnp.exp(a1 * rscale) * okf
                        off2 = off + ((e >> 4) << 4)
                        validm = ((iota + off2) - cs) >> 31
                        dvv = dlist[pl.ds(off2, LANES)] & validm
                        dsel = iota ^ (e & (LANES - 1))
                        sel = dvv & ~((dsel | -dsel) >> 31)
                        for st_ in (8, 4, 2, 1):
                            sel = sel | sel[iota ^ st_]
                        dl0 = sel[0]
                        abase = dl0 * W
                        for t in range(C // LANES):
                            ao = abase + t * LANES
                            acc[pl.ds(ao, LANES)] = (
                                acc[pl.ds(ao, LANES)] +
                                vrows[e, pl.ds(t * LANES, LANES)] * p0)
                        for t in range(C // LANES):
                            ao = abase + C + t * LANES
                            acc[pl.ds(ao, LANES)] = (
                                acc[pl.ds(ao, LANES)] +
                                vrows[e, pl.ds(C + t * LANES, LANES)] * p1)
                        db = dl0 * LANES
                        accden[pl.ds(db, LANES)] = (
                            accden[pl.ds(db, LANES)] + p0 * f0v + p1 * f1v)
                        return 0

                    lax.fori_loop(0, K, edge_body, 0)
                    return 0

                lax.fori_loop(0, nb, drain_body, 0)
                return 0

            lax.fori_loop(0, NSEC, sec_body, 0)
            pltpu.sync_copy(acc, num_out.at[pl.ds(nbase * W, ACC)])
            pltpu.sync_copy(accden, den_out.at[pl.ds(nbase * LANES, ACCD)])
            return 0

        lax.fori_loop(0, RN, round_body, 0)

    zeros = jnp.zeros((ACC,), jnp.float32)
    return body(q, k, v, src, dst, zeros)


def _rgcn_edge_sc(table, tl, g, segs):
    _, W = table.shape
    EM = tl.shape[0]
    NW = NC * NS
    WIN = 512
    RN = -(-segs // (NW * WIN))
    SEGP = NW * WIN * RN
    S = 4096
    K = 64
    NSEC = EM // S
    NV = S // LANES
    ACC = WIN * W
    ACCD = WIN * LANES

    mesh = plsc.VectorSubcoreMesh(core_axis_name="c", subcore_axis_name="s")

    @functools.partial(
        pl.kernel,
        out_type=[jax.ShapeDtypeStruct((SEGP * W,), jnp.float32),
                  jax.ShapeDtypeStruct((SEGP * LANES,), jnp.float32)],
        mesh=mesh,
        scratch_types=[
            pltpu.VMEM((ACC,), jnp.float32),
            pltpu.VMEM((ACCD,), jnp.float32),
            pltpu.VMEM((S,), jnp.int32),
            pltpu.VMEM((S,), jnp.int32),
            pltpu.VMEM((S + LANES,), jnp.int32),
            pltpu.VMEM((S + LANES,), jnp.int32),
            pltpu.VMEM((K, W), jnp.float32),
            pltpu.VMEM((K,), jnp.int32),
            pltpu.SemaphoreType.DMA,
        ],
    )
    def body(tab_hbm, tl_hbm, g_hbm, zeros_hbm,
             sum_out, cnt_out,
             acc, accc, tbuf, gbuf, tlist, glist, trows, tidx, gsem):
        ci = lax.axis_index("c")
        si = lax.axis_index("s")
        wid = ci * NS + si
        iota = lax.iota(jnp.int32, LANES)
        onev = jnp.int32(1) << iota
        onevh = onev << LANES
        f0v = ((((iota | -iota) >> 31) & 1) ^ 1).astype(jnp.float32)
        ziv = jnp.zeros((LANES,), jnp.int32)

        def round_body(r, _0):
            wdw = wid * RN + r
            base = wdw * WIN
            pltpu.sync_copy(zeros_hbm.at[pl.ds(0, ACC)], acc)
            pltpu.sync_copy(zeros_hbm.at[pl.ds(0, ACCD)], accc)

            def sec_body(sec, _1):
                pltpu.sync_copy(tl_hbm.at[pl.ds(sec * S, S)], tbuf)
                pltpu.sync_copy(g_hbm.at[pl.ds(sec * S, S)], gbuf)

                def scan_body(i, carry):
                    cnt, stT, stG = carry
                    g0 = gbuf[pl.ds(i * 32, LANES)]
                    g1 = gbuf[pl.ds(i * 32 + LANES, LANES)]
                    gl0 = g0 - base
                    gl1 = g1 - base
                    neg0 = (gl0 | (WIN - 1 - gl0)) >> 31
                    neg1 = (gl1 | (WIN - 1 - gl1)) >> 31
                    bits = (onev & ~neg0) | (onevh & ~neg1)
                    for st_ in (8, 4, 2, 1):
                        bits = bits | bits[iota ^ st_]
                    bm0 = bits[0]
                    npop = _popcount32(bm0)
                    tv0 = tbuf[pl.ds(i * 32, LANES)]
                    tv1 = tbuf[pl.ds(i * 32 + LANES, LANES)]

                    def ext_body(j, c2):
                        bm, cnt2, sT, sG = c2
                        lane = _ctz(bm)
                        lsp = jnp.full((LANES,), lane & (LANES - 1))
                        him = jnp.full((LANES,), -(lane >> 4))
                        tval = (tv0[lsp] & ~him) | (tv1[lsp] & him)
                        gval = (gl0[lsp] & ~him) | (gl1[lsp] & him)
                        dslot = iota ^ (cnt2 & (LANES - 1))
                        keepm = (dslot | -dslot) >> 31
                        sT = (sT & keepm) | (tval & ~keepm)
                        sG = (sG & keepm) | (gval & ~keepm)

                        @pl.when((cnt2 & (LANES - 1)) == (LANES - 1))
                        def _():
                            tlist[pl.ds(cnt2 - (LANES - 1), LANES)] = sT
                            glist[pl.ds(cnt2 - (LANES - 1), LANES)] = sG

                        return (bm & (bm - 1), cnt2 + 1, sT, sG)

                    bm, cnt, stT, stG = lax.fori_loop(
                        0, npop, ext_body, (bm0, cnt, stT, stG))
                    return (cnt, stT, stG)

                cnt, stT, stG = lax.fori_loop(
                    0, NV // 2, scan_body, (jnp.int32(0), ziv, ziv))

                @pl.when((cnt & (LANES - 1)) != 0)
                def _():
                    fb = (cnt >> 4) << 4
                    tlist[pl.ds(fb, LANES)] = stT
                    glist[pl.ds(fb, LANES)] = stG

                cs = jnp.full((LANES,), cnt)
                nb = (cnt + K - 1) // K

                def drain_body(b, _2):
                    off = b * K
                    for t_ in range(K // LANES):
                        validm = ((iota + off + t_ * LANES) - cs) >> 31
                        tv = tlist[pl.ds(off + t_ * LANES, LANES)] & validm
                        tidx[pl.ds(t_ * LANES, LANES)] = tv
                    pltpu.async_copy(tab_hbm.at[tidx], trows, gsem).wait()

                    def edge_body(e, _3):
                        okf = jnp.full(
                            (LANES,),
                            ((((off + e) - cnt) >> 31) & 1).astype(jnp.float32))
                        off2 = off + ((e >> 4) << 4)
                        validm2 = ((iota + off2) - cs) >> 31
                        gvv = glist[pl.ds(off2, LANES)] & validm2
                        dsel = iota ^ (e & (LANES - 1))
                        sel = gvv & ~((dsel | -dsel) >> 31)
                        for st_ in (8, 4, 2, 1):
                            sel = sel | sel[iota ^ st_]
                        gl0 = sel[0]
                        abase = gl0 * W
                        for t in range(W // LANES):
                            ao = abase + t * LANES
                            acc[pl.ds(ao, LANES)] = (
                                acc[pl.ds(ao, LANES)] +
                                trows[e, pl.ds(t * LANES, LANES)] * okf)
                        cb = gl0 * LANES
                        accc[pl.ds(cb, LANES)] = (
                            accc[pl.ds(cb, LANES)] + okf * f0v)
                        return 0

                    lax.fori_loop(0, K, edge_body, 0)
                    return 0

                lax.fori_loop(0, nb, drain_body, 0)
                return 0

            lax.fori_loop(0, NSEC, sec_body, 0)
            pltpu.sync_copy(acc, sum_out.at[pl.ds(base * W, ACC)])
            pltpu.sync_copy(accc, cnt_out.at[pl.ds(base * LANES, ACCD)])
            return 0

        lax.fori_loop(0, RN, round_body, 0)

    zeros = jnp.zeros((ACC,), jnp.float32)
    sums, cnts = body(table, tl, g, zeros)
    return sums, cnts, SEGP


def kernel(body, face, r_hand, l_hand, ecg, flow, params, pose_batch_edge_index, pose_batch_vector, batch_edge_index, batch_edge_types):
    pf = params['pf']
    mf = params['mf']
    B, C = body.shape
    pose = jnp.stack([body, face, r_hand, l_hand], axis=1)
    n = B * 4
    x = pose.reshape(n, C)

    src, dst = pose_batch_edge_index[0], pose_batch_edge_index[1]
    q = x @ pf['tqW'] + pf['tqb']
    k = x @ pf['tkW'] + pf['tkb']
    v = x @ pf['tvW'] + pf['tvb']
    numf, denf = _tconv_edge_sc(q, k, v, src, dst)
    num = numf.reshape(n, H * C)
    den = denf.reshape(n, LANES)[:, :H]
    out = num.reshape(n, H, C) / (den[:, :, None] + 1e-16)
    pfx = out.reshape(n, H * C) + x @ pf['tsW'] + pf['tsb']
    pfx = jax.nn.relu(_ln(pfx, pf['n1g'], pf['n1b'])).reshape(B, 4, H * C)

    conf = jax.nn.sigmoid(jax.nn.relu(pose @ pf['cW1'] + pf['cb1']) @ pf['cW2'] + pf['cb2'])
    flat = (pfx * conf).reshape(B, -1)
    pooled = jax.nn.relu(flat @ pf['apW'] + pf['apb'])
    pooled = jax.nn.relu(_ln(pooled, pf['n2g'], pf['n2b']))
    fused = pooled @ pf['mlpW'] + pf['mlpb']
    cls = pooled @ pf['clsW'] + pf['clsb']

    xm = jnp.stack([ecg, flow, fused], axis=1)
    cp = mf['cma']
    qc = xm @ cp['Wq'] + cp['bq']
    kc = xm @ cp['Wk'] + cp['bk']
    vc = xm @ cp['Wv'] + cp['bv']
    attn = jax.nn.softmax(jnp.einsum('bnc,bmc->bnm', qc, kc) / np.sqrt(C), axis=-1)
    co = jnp.einsum('bnm,bmc->bnc', attn, vc)
    gate = jax.nn.sigmoid(jnp.concatenate([co, xm], axis=-1) @ cp['Wg'] + cp['bg'])
    vx = _ln(gate * co + (1.0 - gate) * xm, cp['ln_g'], cp['ln_b'])

    conf2 = jax.nn.sigmoid(jax.nn.relu(vx @ mf['cW1'] + mf['cb1']) @ mf['cW2'] + mf['cb2'])
    wx = vx * conf2
    x2 = jax.nn.relu(jnp.concatenate([xm, wx], axis=-1) @ mf['fmW'] + mf['fmb'])
    xn = _ln(x2, mf['nbg'], mf['nbb'])
    xnf = xn.reshape(-1, C)
    nm = xnf.shape[0]

    rg = mf['rgcn']
    rootp = xnf @ rg['root'] + rg['bias']
    t = jnp.einsum('nc,rcd->rnd', xnf, rg['W'])
    src2, dst2 = batch_edge_index[0], batch_edge_index[1]
    tl = batch_edge_types * nm + src2
    g = batch_edge_types * nm + dst2
    sums, cnts, segp = _rgcn_edge_sc(t.reshape(R * nm, C), tl, g, R * nm)
    s = sums.reshape(segp, C)[:R * nm].reshape(R, nm, C)
    cnt = cnts.reshape(segp, LANES)[:R * nm, :1].reshape(R, nm, 1)
    xr = rootp + (s / jnp.maximum(cnt, 1.0)).sum(0)
    xr = jax.nn.relu(_ln(xr.reshape(B, 3, C), mf['nag'], mf['nab']))

    den2 = jnp.maximum(conf2.sum(1), 1e-8)
    pooled2 = (xr * conf2).sum(1) / den2
    logits = pooled2 @ mf['headW'] + mf['headb']
    return cls, logits


# speedup vs baseline: 2.2984x; 2.2984x over previous
import functools

import jax
import jax.numpy as jnp
import numpy as np
from jax import lax
from jax.experimental import pallas as pl
from jax.experimental.pallas import tpu as pltpu
from jax.experimental.pallas import tpu_sc as plsc

H = 2
R = 3
LANES = 16
NC = 2
NS = 16


def _ln(x, g, b, eps=1e-5):
    m = x.mean(-1, keepdims=True)
    v = ((x - m) ** 2).mean(-1, keepdims=True)
    return (x - m) / jnp.sqrt(v + eps) * g + b


def _popcount32(bm):
    x = bm - ((bm >> 1) & 0x55555555)
    x = (x & 0x33333333) + ((x >> 2) & 0x33333333)
    x = (x + (x >> 4)) & 0x0F0F0F0F
    return (x + (x >> 8) + (x >> 16) + (x >> 24)) & 0x3F


def _ctz(bm):
    low = bm & (-bm)
    return (((lax.bitcast_convert_type(low.astype(jnp.float32), jnp.int32)
              >> 23) & 0xFF) - 127)


def _tconv_edge_sc(q, k, v, src, dst):
    n, W = q.shape
    EP = src.shape[0]
    C = W // H
    NW = NC * NS
    WIN = 256
    RN = n // (NW * WIN)
    S = 8192
    K = 16
    NSEC = EP // S
    NV = S // LANES
    ACC = WIN * W
    ACCD = WIN * LANES
    rscale = 1.0 / np.sqrt(C)

    mesh = plsc.VectorSubcoreMesh(core_axis_name="c", subcore_axis_name="s")

    @functools.partial(
        pl.kernel,
        out_type=[jax.ShapeDtypeStruct((n * W,), jnp.float32),
                  jax.ShapeDtypeStruct((n * LANES,), jnp.float32)],
        mesh=mesh,
        scratch_types=[
            pltpu.VMEM((ACC,), jnp.float32),
            pltpu.VMEM((ACCD,), jnp.float32),
            pltpu.VMEM((S,), jnp.int32),
            pltpu.VMEM((S,), jnp.int32),
            pltpu.VMEM((S + LANES,), jnp.int32),
            pltpu.VMEM((S + LANES,), jnp.int32),
            pltpu.VMEM((K, W), jnp.float32),
            pltpu.VMEM((K, W), jnp.float32),
            pltpu.VMEM((K, W), jnp.float32),
            pltpu.VMEM((K,), jnp.int32),
            pltpu.VMEM((K,), jnp.int32),
            pltpu.SemaphoreType.DMA,
        ],
    )
    def body(q_hbm, k_hbm, v_hbm, src_hbm, dst_hbm, zeros_hbm,
             num_out, den_out,
             acc, accden, sbuf, dbuf, slist, dlist,
             qrows, krows, vrows, sidx, gidx, gsem):
        ci = lax.axis_index("c")
        si = lax.axis_index("s")
        wid = ci * NS + si
        iota = lax.iota(jnp.int32, LANES)
        onev = jnp.int32(1) << iota
        onevh = onev << LANES
        zv = jnp.zeros((LANES,), jnp.float32)
        ziv = jnp.zeros((LANES,), jnp.int32)
        d1 = iota ^ 1
        f0v = ((((iota | -iota) >> 31) & 1) ^ 1).astype(jnp.float32)
        f1v = ((((d1 | -d1) >> 31) & 1) ^ 1).astype(jnp.float32)

        def round_body(r, _0):
            wdw = wid * RN + r
            nbase = wdw * WIN
            pltpu.sync_copy(zeros_hbm.at[pl.ds(0, ACC)], acc)
            pltpu.sync_copy(zeros_hbm.at[pl.ds(0, ACCD)], accden)

            def sec_body(sec, _1):
                pltpu.sync_copy(src_hbm.at[pl.ds(sec * S, S)], sbuf)
                pltpu.sync_copy(dst_hbm.at[pl.ds(sec * S, S)], dbuf)

                def scan_body(i, carry):
                    cnt, stS, stD = carry
                    d0 = dbuf[pl.ds(i * 32, LANES)]
                    d1 = dbuf[pl.ds(i * 32 + LANES, LANES)]
                    dl0 = d0 - nbase
                    dl1 = d1 - nbase
                    neg0 = (dl0 | (WIN - 1 - dl0)) >> 31
                    neg1 = (dl1 | (WIN - 1 - dl1)) >> 31
                    bits = (onev & ~neg0) | (onevh & ~neg1)
                    for st_ in (8, 4, 2, 1):
                        bits = bits | bits[iota ^ st_]
                    bm0 = bits[0]
                    npop = _popcount32(bm0)
                    sv0 = sbuf[pl.ds(i * 32, LANES)]
                    sv1 = sbuf[pl.ds(i * 32 + LANES, LANES)]

                    def ext_body(j, c2):
                        bm, cnt2, sS, sD = c2
                        lane = _ctz(bm)
                        lsp = jnp.full((LANES,), lane & (LANES - 1))
                        him = jnp.full((LANES,), -(lane >> 4))
                        sval = (sv0[lsp] & ~him) | (sv1[lsp] & him)
                        dval = (dl0[lsp] & ~him) | (dl1[lsp] & him)
                        dslot = iota ^ (cnt2 & (LANES - 1))
                        keepm = (dslot | -dslot) >> 31
                        sS = (sS & keepm) | (sval & ~keepm)
                        sD = (sD & keepm) | (dval & ~keepm)

                        @pl.when((cnt2 & (LANES - 1)) == (LANES - 1))
                        def _():
                            slist[pl.ds(cnt2 - (LANES - 1), LANES)] = sS
                            dlist[pl.ds(cnt2 - (LANES - 1), LANES)] = sD

                        return (bm & (bm - 1), cnt2 + 1, sS, sD)

                    bm, cnt, stS, stD = lax.fori_loop(
                        0, npop, ext_body, (bm0, cnt, stS, stD))
                    return (cnt, stS, stD)

                cnt, stS, stD = lax.fori_loop(
                    0, NV // 2, scan_body, (jnp.int32(0), ziv, ziv))

                @pl.when((cnt & (LANES - 1)) != 0)
                def _():
                    fb = (cnt >> 4) << 4
                    slist[pl.ds(fb, LANES)] = stS
                    dlist[pl.ds(fb, LANES)] = stD

                cs = jnp.full((LANES,), cnt)
                nb = (cnt + K - 1) // K

                def drain_body(b, _2):
                    off = b * K
                    for t_ in range(K // LANES):
                        validm = ((iota + off + t_ * LANES) - cs) >> 31
                        sv = slist[pl.ds(off + t_ * LANES, LANES)] & validm
                        dv = dlist[pl.ds(off + t_ * LANES, LANES)] & validm
                        sidx[pl.ds(t_ * LANES, LANES)] = sv
                        gidx[pl.ds(t_ * LANES, LANES)] = dv + (nbase & validm)
                    cq = pltpu.async_copy(q_hbm.at[gidx], qrows, gsem)
                    ck = pltpu.async_copy(k_hbm.at[sidx], krows, gsem)
                    cv = pltpu.async_copy(v_hbm.at[sidx], vrows, gsem)
                    cq.wait()
                    ck.wait()
                    cv.wait()

                    def edge_body(e, _3):
                        okf = jnp.full(
                            (LANES,),
                            ((((off + e) - cnt) >> 31) & 1).astype(jnp.float32))
                        a0 = qrows[e, pl.ds(0, LANES)] * krows[e, pl.ds(0, LANES)]
                        a1 = qrows[e, pl.ds(C, LANES)] * krows[e, pl.ds(C, LANES)]
                        for t in range(1, C // LANES):
                            a0 += (qrows[e, pl.ds(t * LANES, LANES)] *
                                   krows[e, pl.ds(t * LANES, LANES)])
                            a1 += (qrows[e, pl.ds(C + t * LANES, LANES)] *
                                   krows[e, pl.ds(C + t * LANES, LANES)])
                        for st_ in (8, 4, 2, 1):
                            a0 = a0 + a0[iota ^ st_]
                            a1 = a1 + a1[iota ^ st_]
                        p0 = jnp.exp(a0 * rscale) * okf
                        p1 = jnp.exp(a1 * rscale) * okf
                        off2 = off + ((e >> 4) << 4)
                        validm = ((iota + off2) - cs) >> 31
                        dvv = dlist[pl.ds(off2, LANES)] & validm
                        dsel = iota ^ (e & (LANES - 1))
                        sel = dvv & ~((dsel | -dsel) >> 31)
                        for st_ in (8, 4, 2, 1):
                            sel = sel | sel[iota ^ st_]
                        dl0 = sel[0]
                        abase = dl0 * W
                        for t in range(C // LANES):
                            ao = abase + t * LANES
                            acc[pl.ds(ao, LANES)] = (
                                acc[pl.ds(ao, LANES)] +
                                vrows[e, pl.ds(t * LANES, LANES)] * p0)
                        for t in range(C // LANES):
                            ao = abase + C + t * LANES
                            acc[pl.ds(ao, LANES)] = (
                                acc[pl.ds(ao, LANES)] +
                                vrows[e, pl.ds(C + t * LANES, LANES)] * p1)
                        db = dl0 * LANES
                        accden[pl.ds(db, LANES)] = (
                            accden[pl.ds(db, LANES)] + p0 * f0v + p1 * f1v)
                        return 0

                    lax.fori_loop(0, K, edge_body, 0)
                    return 0

                lax.fori_loop(0, nb, drain_body, 0)
                return 0

            lax.fori_loop(0, NSEC, sec_body, 0)
            pltpu.sync_copy(acc, num_out.at[pl.ds(nbase * W, ACC)])
            pltpu.sync_copy(accden, den_out.at[pl.ds(nbase * LANES, ACCD)])
            return 0

        lax.fori_loop(0, RN, round_body, 0)

    zeros = jnp.zeros((ACC,), jnp.float32)
    return body(q, k, v, src, dst, zeros)


def _rgcn_edge_sc(table, tl, g, segs):
    _, W = table.shape
    EM = tl.shape[0]
    NW = NC * NS
    WIN = 512
    RN = -(-segs // (NW * WIN))
    SEGP = NW * WIN * RN
    S = 8192
    K = 16
    NSEC = EM // S
    NV = S // LANES
    ACC = WIN * W
    ACCD = WIN * LANES

    mesh = plsc.VectorSubcoreMesh(core_axis_name="c", subcore_axis_name="s")

    @functools.partial(
        pl.kernel,
        out_type=[jax.ShapeDtypeStruct((SEGP * W,), jnp.float32),
                  jax.ShapeDtypeStruct((SEGP * LANES,), jnp.float32)],
        mesh=mesh,
        scratch_types=[
            pltpu.VMEM((ACC,), jnp.float32),
            pltpu.VMEM((ACCD,), jnp.float32),
            pltpu.VMEM((S,), jnp.int32),
            pltpu.VMEM((S,), jnp.int32),
            pltpu.VMEM((S + LANES,), jnp.int32),
            pltpu.VMEM((S + LANES,), jnp.int32),
            pltpu.VMEM((K, W), jnp.float32),
            pltpu.VMEM((K,), jnp.int32),
            pltpu.SemaphoreType.DMA,
        ],
    )
    def body(tab_hbm, tl_hbm, g_hbm, zeros_hbm,
             sum_out, cnt_out,
             acc, accc, tbuf, gbuf, tlist, glist, trows, tidx, gsem):
        ci = lax.axis_index("c")
        si = lax.axis_index("s")
        wid = ci * NS + si
        iota = lax.iota(jnp.int32, LANES)
        onev = jnp.int32(1) << iota
        onevh = onev << LANES
        f0v = ((((iota | -iota) >> 31) & 1) ^ 1).astype(jnp.float32)
        ziv = jnp.zeros((LANES,), jnp.int32)

        def round_body(r, _0):
            wdw = wid * RN + r
            base = wdw * WIN
            pltpu.sync_copy(zeros_hbm.at[pl.ds(0, ACC)], acc)
            pltpu.sync_copy(zeros_hbm.at[pl.ds(0, ACCD)], accc)

            def sec_body(sec, _1):
                pltpu.sync_copy(tl_hbm.at[pl.ds(sec * S, S)], tbuf)
                pltpu.sync_copy(g_hbm.at[pl.ds(sec * S, S)], gbuf)

                def scan_body(i, carry):
                    cnt, stT, stG = carry
                    g0 = gbuf[pl.ds(i * 32, LANES)]
                    g1 = gbuf[pl.ds(i * 32 + LANES, LANES)]
                    gl0 = g0 - base
                    gl1 = g1 - base
                    neg0 = (gl0 | (WIN - 1 - gl0)) >> 31
                    neg1 = (gl1 | (WIN - 1 - gl1)) >> 31
                    bits = (onev & ~neg0) | (onevh & ~neg1)
                    for st_ in (8, 4, 2, 1):
                        bits = bits | bits[iota ^ st_]
                    bm0 = bits[0]
                    npop = _popcount32(bm0)
                    tv0 = tbuf[pl.ds(i * 32, LANES)]
                    tv1 = tbuf[pl.ds(i * 32 + LANES, LANES)]

                    def ext_body(j, c2):
                        bm, cnt2, sT, sG = c2
                        lane = _ctz(bm)
                        lsp = jnp.full((LANES,), lane & (LANES - 1))
                        him = jnp.full((LANES,), -(lane >> 4))
                        tval = (tv0[lsp] & ~him) | (tv1[lsp] & him)
                        gval = (gl0[lsp] & ~him) | (gl1[lsp] & him)
                        dslot = iota ^ (cnt2 & (LANES - 1))
                        keepm = (dslot | -dslot) >> 31
                        sT = (sT & keepm) | (tval & ~keepm)
                        sG = (sG & keepm) | (gval & ~keepm)

                        @pl.when((cnt2 & (LANES - 1)) == (LANES - 1))
                        def _():
                            tlist[pl.ds(cnt2 - (LANES - 1), LANES)] = sT
                            glist[pl.ds(cnt2 - (LANES - 1), LANES)] = sG

                        return (bm & (bm - 1), cnt2 + 1, sT, sG)

                    bm, cnt, stT, stG = lax.fori_loop(
                        0, npop, ext_body, (bm0, cnt, stT, stG))
                    return (cnt, stT, stG)

                cnt, stT, stG = lax.fori_loop(
                    0, NV // 2, scan_body, (jnp.int32(0), ziv, ziv))

                @pl.when((cnt & (LANES - 1)) != 0)
                def _():
                    fb = (cnt >> 4) << 4
                    tlist[pl.ds(fb, LANES)] = stT
                    glist[pl.ds(fb, LANES)] = stG

                cs = jnp.full((LANES,), cnt)
                nb = (cnt + K - 1) // K

                def drain_body(b, _2):
                    off = b * K
                    for t_ in range(K // LANES):
                        validm = ((iota + off + t_ * LANES) - cs) >> 31
                        tv = tlist[pl.ds(off + t_ * LANES, LANES)] & validm
                        tidx[pl.ds(t_ * LANES, LANES)] = tv
                    pltpu.async_copy(tab_hbm.at[tidx], trows, gsem).wait()

                    def edge_body(e, _3):
                        okf = jnp.full(
                            (LANES,),
                            ((((off + e) - cnt) >> 31) & 1).astype(jnp.float32))
                        off2 = off + ((e >> 4) << 4)
                        validm2 = ((iota + off2) - cs) >> 31
                        gvv = glist[pl.ds(off2, LANES)] & validm2
                        dsel = iota ^ (e & (LANES - 1))
                        sel = gvv & ~((dsel | -dsel) >> 31)
                        for st_ in (8, 4, 2, 1):
                            sel = sel | sel[iota ^ st_]
                        gl0 = sel[0]
                        abase = gl0 * W
                        for t in range(W // LANES):
                            ao = abase + t * LANES
                            acc[pl.ds(ao, LANES)] = (
                                acc[pl.ds(ao, LANES)] +
                                trows[e, pl.ds(t * LANES, LANES)] * okf)
                        cb = gl0 * LANES
                        accc[pl.ds(cb, LANES)] = (
                            accc[pl.ds(cb, LANES)] + okf * f0v)
                        return 0

                    lax.fori_loop(0, K, edge_body, 0)
                    return 0

                lax.fori_loop(0, nb, drain_body, 0)
                return 0

            lax.fori_loop(0, NSEC, sec_body, 0)
            pltpu.sync_copy(acc, sum_out.at[pl.ds(base * W, ACC)])
            pltpu.sync_copy(accc, cnt_out.at[pl.ds(base * LANES, ACCD)])
            return 0

        lax.fori_loop(0, RN, round_body, 0)

    zeros = jnp.zeros((ACC,), jnp.float32)
    sums, cnts = body(table, tl, g, zeros)
    return sums, cnts, SEGP


def kernel(body, face, r_hand, l_hand, ecg, flow, params, pose_batch_edge_index, pose_batch_vector, batch_edge_index, batch_edge_types):
    pf = params['pf']
    mf = params['mf']
    B, C = body.shape
    pose = jnp.stack([body, face, r_hand, l_hand], axis=1)
    n = B * 4
    x = pose.reshape(n, C)

    src, dst = pose_batch_edge_index[0], pose_batch_edge_index[1]
    q = x @ pf['tqW'] + pf['tqb']
    k = x @ pf['tkW'] + pf['tkb']
    v = x @ pf['tvW'] + pf['tvb']
    numf, denf = _tconv_edge_sc(q, k, v, src, dst)
    num = numf.reshape(n, H * C)
    den = denf.reshape(n, LANES)[:, :H]
    out = num.reshape(n, H, C) / (den[:, :, None] + 1e-16)
    pfx = out.reshape(n, H * C) + x @ pf['tsW'] + pf['tsb']
    pfx = jax.nn.relu(_ln(pfx, pf['n1g'], pf['n1b'])).reshape(B, 4, H * C)

    conf = jax.nn.sigmoid(jax.nn.relu(pose @ pf['cW1'] + pf['cb1']) @ pf['cW2'] + pf['cb2'])
    flat = (pfx * conf).reshape(B, -1)
    pooled = jax.nn.relu(flat @ pf['apW'] + pf['apb'])
    pooled = jax.nn.relu(_ln(pooled, pf['n2g'], pf['n2b']))
    fused = pooled @ pf['mlpW'] + pf['mlpb']
    cls = pooled @ pf['clsW'] + pf['clsb']

    xm = jnp.stack([ecg, flow, fused], axis=1)
    cp = mf['cma']
    qc = xm @ cp['Wq'] + cp['bq']
    kc = xm @ cp['Wk'] + cp['bk']
    vc = xm @ cp['Wv'] + cp['bv']
    attn = jax.nn.softmax(jnp.einsum('bnc,bmc->bnm', qc, kc) / np.sqrt(C), axis=-1)
    co = jnp.einsum('bnm,bmc->bnc', attn, vc)
    gate = jax.nn.sigmoid(jnp.concatenate([co, xm], axis=-1) @ cp['Wg'] + cp['bg'])
    vx = _ln(gate * co + (1.0 - gate) * xm, cp['ln_g'], cp['ln_b'])

    conf2 = jax.nn.sigmoid(jax.nn.relu(vx @ mf['cW1'] + mf['cb1']) @ mf['cW2'] + mf['cb2'])
    wx = vx * conf2
    x2 = jax.nn.relu(jnp.concatenate([xm, wx], axis=-1) @ mf['fmW'] + mf['fmb'])
    xn = _ln(x2, mf['nbg'], mf['nbb'])
    xnf = xn.reshape(-1, C)
    nm = xnf.shape[0]

    rg = mf['rgcn']
    rootp = xnf @ rg['root'] + rg['bias']
    t = jnp.einsum('nc,rcd->rnd', xnf, rg['W'])
    src2, dst2 = batch_edge_index[0], batch_edge_index[1]
    tl = batch_edge_types * nm + src2
    g = batch_edge_types * nm + dst2
    sums, cnts, segp = _rgcn_edge_sc(t.reshape(R * nm, C), tl, g, R * nm)
    s = sums.reshape(segp, C)[:R * nm].reshape(R, nm, C)
    cnt = cnts.reshape(segp, LANES)[:R * nm, :1].reshape(R, nm, 1)
    xr = rootp + (s / jnp.maximum(cnt, 1.0)).sum(0)
    xr = jax.nn.relu(_ln(xr.reshape(B, 3, C), mf['nag'], mf['nab']))

    den2 = jnp.maximum(conf2.sum(1), 1e-8)
    pooled2 = (xr * conf2).sum(1) / den2
    logits = pooled2 @ mf['headW'] + mf['headb']
    return cls, logits


# speedup vs baseline: 2.3002x; 1.0008x over previous
import functools

import jax
import jax.numpy as jnp
import numpy as np
from jax import lax
from jax.experimental import pallas as pl
from jax.experimental.pallas import tpu as pltpu
from jax.experimental.pallas import tpu_sc as plsc

H = 2
R = 3
LANES = 16
NC = 2
NS = 16


def _ln(x, g, b, eps=1e-5):
    m = x.mean(-1, keepdims=True)
    v = ((x - m) ** 2).mean(-1, keepdims=True)
    return (x - m) / jnp.sqrt(v + eps) * g + b


def _popcount32(bm):
    x = bm - ((bm >> 1) & 0x55555555)
    x = (x & 0x33333333) + ((x >> 2) & 0x33333333)
    x = (x + (x >> 4)) & 0x0F0F0F0F
    return (x + (x >> 8) + (x >> 16) + (x >> 24)) & 0x3F


def _ctz(bm):
    low = bm & (-bm)
    return (((lax.bitcast_convert_type(low.astype(jnp.float32), jnp.int32)
              >> 23) & 0xFF) - 127)


def _proj_tc(x, w, b):
    n, c = x.shape
    m = w.shape[1]
    BN = 1024

    def body(x_ref, w_ref, b_ref, o_ref):
        o_ref[...] = (
            jnp.dot(x_ref[...], w_ref[...],
                    preferred_element_type=jnp.float32) + b_ref[...])

    return pl.pallas_call(
        body,
        grid=(n // BN,),
        in_specs=[pl.BlockSpec((BN, c), lambda i: (i, 0)),
                  pl.BlockSpec((c, m), lambda i: (0, 0)),
                  pl.BlockSpec((1, m), lambda i: (0, 0))],
        out_specs=pl.BlockSpec((BN, m), lambda i: (i, 0)),
        out_shape=jax.ShapeDtypeStruct((n, m), jnp.float32),
    )(x, w, b.reshape(1, m))


def _tconv_edge_sc(q, k, v, src, dst):
    n, W = q.shape
    EP = src.shape[0]
    C = W // H
    NW = NC * NS
    WIN = 256
    RN = n // (NW * WIN)
    S = 8192
    K = 16
    NSEC = EP // S
    NV = S // LANES
    ACC = WIN * W
    ACCD = WIN * LANES
    rscale = 1.0 / np.sqrt(C)

    mesh = plsc.VectorSubcoreMesh(core_axis_name="c", subcore_axis_name="s")

    @functools.partial(
        pl.kernel,
        out_type=[jax.ShapeDtypeStruct((n * W,), jnp.float32),
                  jax.ShapeDtypeStruct((n * LANES,), jnp.float32)],
        mesh=mesh,
        scratch_types=[
            pltpu.VMEM((ACC,), jnp.float32),
            pltpu.VMEM((ACCD,), jnp.float32),
            pltpu.VMEM((S,), jnp.int32),
            pltpu.VMEM((S,), jnp.int32),
            pltpu.VMEM((S + LANES,), jnp.int32),
            pltpu.VMEM((S + LANES,), jnp.int32),
            pltpu.VMEM((K, W), jnp.float32),
            pltpu.VMEM((K, W), jnp.float32),
            pltpu.VMEM((K, W), jnp.float32),
            pltpu.VMEM((K,), jnp.int32),
            pltpu.VMEM((K,), jnp.int32),
            pltpu.SemaphoreType.DMA,
        ],
    )
    def body(q_hbm, k_hbm, v_hbm, src_hbm, dst_hbm, zeros_hbm,
             num_out, den_out,
             acc, accden, sbuf, dbuf, slist, dlist,
             qrows, krows, vrows, sidx, gidx, gsem):
        ci = lax.axis_index("c")
        si = lax.axis_index("s")
        wid = ci * NS + si
        iota = lax.iota(jnp.int32, LANES)
        onev = jnp.int32(1) << iota
        onevh = onev << LANES
        zv = jnp.zeros((LANES,), jnp.float32)
        ziv = jnp.zeros((LANES,), jnp.int32)
        d1 = iota ^ 1
        f0v = ((((iota | -iota) >> 31) & 1) ^ 1).astype(jnp.float32)
        f1v = ((((d1 | -d1) >> 31) & 1) ^ 1).astype(jnp.float32)

        def round_body(r, _0):
            wdw = wid * RN + r
            nbase = wdw * WIN
            pltpu.sync_copy(zeros_hbm.at[pl.ds(0, ACC)], acc)
            pltpu.sync_copy(zeros_hbm.at[pl.ds(0, ACCD)], accden)

            def sec_body(sec, _1):
                pltpu.sync_copy(src_hbm.at[pl.ds(sec * S, S)], sbuf)
                pltpu.sync_copy(dst_hbm.at[pl.ds(sec * S, S)], dbuf)

                def scan_body(i, carry):
                    cnt, stS, stD = carry
                    d0 = dbuf[pl.ds(i * 32, LANES)]
                    d1 = dbuf[pl.ds(i * 32 + LANES, LANES)]
                    dl0 = d0 - nbase
                    dl1 = d1 - nbase
                    neg0 = (dl0 | (WIN - 1 - dl0)) >> 31
                    neg1 = (dl1 | (WIN - 1 - dl1)) >> 31
                    bits = (onev & ~neg0) | (onevh & ~neg1)
                    for st_ in (8, 4, 2, 1):
                        bits = bits | bits[iota ^ st_]
                    bm0 = bits[0]
                    npop = _popcount32(bm0)
                    sv0 = sbuf[pl.ds(i * 32, LANES)]
                    sv1 = sbuf[pl.ds(i * 32 + LANES, LANES)]

                    def ext_body(j, c2):
                        bm, cnt2, sS, sD = c2
                        lane = _ctz(bm)
                        lsp = jnp.full((LANES,), lane & (LANES - 1))
                        him = jnp.full((LANES,), -(lane >> 4))
                        sval = (sv0[lsp] & ~him) | (sv1[lsp] & him)
                        dval = (dl0[lsp] & ~him) | (dl1[lsp] & him)
                        dslot = iota ^ (cnt2 & (LANES - 1))
                        keepm = (dslot | -dslot) >> 31
                        sS = (sS & keepm) | (sval & ~keepm)
                        sD = (sD & keepm) | (dval & ~keepm)

                        @pl.when((cnt2 & (LANES - 1)) == (LANES - 1))
                        def _():
                            slist[pl.ds(cnt2 - (LANES - 1), LANES)] = sS
                            dlist[pl.ds(cnt2 - (LANES - 1), LANES)] = sD

                        return (bm & (bm - 1), cnt2 + 1, sS, sD)

                    bm, cnt, stS, stD = lax.fori_loop(
                        0, npop, ext_body, (bm0, cnt, stS, stD))
                    return (cnt, stS, stD)

                cnt, stS, stD = lax.fori_loop(
                    0, NV // 2, scan_body, (jnp.int32(0), ziv, ziv))

                @pl.when((cnt & (LANES - 1)) != 0)
                def _():
                    fb = (cnt >> 4) << 4
                    slist[pl.ds(fb, LANES)] = stS
                    dlist[pl.ds(fb, LANES)] = stD

                cs = jnp.full((LANES,), cnt)
                nb = (cnt + K - 1) // K

                def drain_body(b, _2):
                    off = b * K
                    for t_ in range(K // LANES):
                        validm = ((iota + off + t_ * LANES) - cs) >> 31
                        sv = slist[pl.ds(off + t_ * LANES, LANES)] & validm
                        dv = dlist[pl.ds(off + t_ * LANES, LANES)] & validm
                        sidx[pl.ds(t_ * LANES, LANES)] = sv
                        gidx[pl.ds(t_ * LANES, LANES)] = dv + (nbase & validm)
                    cq = pltpu.async_copy(q_hbm.at[gidx], qrows, gsem)
                    ck = pltpu.async_copy(k_hbm.at[sidx], krows, gsem)
                    cv = pltpu.async_copy(v_hbm.at[sidx], vrows, gsem)
                    cq.wait()
                    ck.wait()
                    cv.wait()

                    def edge_body(e, _3):
                        okf = jnp.full(
                            (LANES,),
                            ((((off + e) - cnt) >> 31) & 1).astype(jnp.float32))
                        a0 = qrows[e, pl.ds(0, LANES)] * krows[e, pl.ds(0, LANES)]
                        a1 = qrows[e, pl.ds(C, LANES)] * krows[e, pl.ds(C, LANES)]
                        for t in range(1, C // LANES):
                            a0 += (qrows[e, pl.ds(t * LANES, LANES)] *
                                   krows[e, pl.ds(t * LANES, LANES)])
                            a1 += (qrows[e, pl.ds(C + t * LANES, LANES)] *
                                   krows[e, pl.ds(C + t * LANES, LANES)])
                        for st_ in (8, 4, 2, 1):
                            a0 = a0 + a0[iota ^ st_]
                            a1 = a1 + a1[iota ^ st_]
                        p0 = jnp.exp(a0 * rscale) * okf
                        p1 = jnp.exp(a1 * rscale) * okf
                        off2 = off + ((e >> 4) << 4)
                        validm = ((iota + off2) - cs) >> 31
                        dvv = dlist[pl.ds(off2, LANES)] & validm
                        dsel = iota ^ (e & (LANES - 1))
                        sel = dvv & ~((dsel | -dsel) >> 31)
                        for st_ in (8, 4, 2, 1):
                            sel = sel | sel[iota ^ st_]
                        dl0 = sel[0]
                        abase = dl0 * W
                        for t in range(C // LANES):
                            ao = abase + t * LANES
                            acc[pl.ds(ao, LANES)] = (
                                acc[pl.ds(ao, LANES)] +
                                vrows[e, pl.ds(t * LANES, LANES)] * p0)
                        for t in range(C // LANES):
                            ao = abase + C + t * LANES
                            acc[pl.ds(ao, LANES)] = (
                                acc[pl.ds(ao, LANES)] +
                                vrows[e, pl.ds(C + t * LANES, LANES)] * p1)
                        db = dl0 * LANES
                        accden[pl.ds(db, LANES)] = (
                            accden[pl.ds(db, LANES)] + p0 * f0v + p1 * f1v)
                        return 0

                    lax.fori_loop(0, K, edge_body, 0)
                    return 0

                lax.fori_loop(0, nb, drain_body, 0)
                return 0

            lax.fori_loop(0, NSEC, sec_body, 0)
            pltpu.sync_copy(acc, num_out.at[pl.ds(nbase * W, ACC)])
            pltpu.sync_copy(accden, den_out.at[pl.ds(nbase * LANES, ACCD)])
            return 0

        lax.fori_loop(0, RN, round_body, 0)

    zeros = jnp.zeros((ACC,), jnp.float32)
    return body(q, k, v, src, dst, zeros)


def _rgcn_edge_sc(table, tl, g, segs):
    _, W = table.shape
    EM = tl.shape[0]
    NW = NC * NS
    WIN = 512
    RN = -(-segs // (NW * WIN))
    SEGP = NW * WIN * RN
    S = 8192
    K = 16
    NSEC = EM // S
    NV = S // LANES
    ACC = WIN * W
    ACCD = WIN * LANES

    mesh = plsc.VectorSubcoreMesh(core_axis_name="c", subcore_axis_name="s")

    @functools.partial(
        pl.kernel,
        out_type=[jax.ShapeDtypeStruct((SEGP * W,), jnp.float32),
                  jax.ShapeDtypeStruct((SEGP * LANES,), jnp.float32)],
        mesh=mesh,
        scratch_types=[
            pltpu.VMEM((ACC,), jnp.float32),
            pltpu.VMEM((ACCD,), jnp.float32),
            pltpu.VMEM((S,), jnp.int32),
            pltpu.VMEM((S,), jnp.int32),
            pltpu.VMEM((S + LANES,), jnp.int32),
            pltpu.VMEM((S + LANES,), jnp.int32),
            pltpu.VMEM((K, W), jnp.float32),
            pltpu.VMEM((K,), jnp.int32),
            pltpu.SemaphoreType.DMA,
        ],
    )
    def body(tab_hbm, tl_hbm, g_hbm, zeros_hbm,
             sum_out, cnt_out,
             acc, accc, tbuf, gbuf, tlist, glist, trows, tidx, gsem):
        ci = lax.axis_index("c")
        si = lax.axis_index("s")
        wid = ci * NS + si
        iota = lax.iota(jnp.int32, LANES)
        onev = jnp.int32(1) << iota
        onevh = onev << LANES
        f0v = ((((iota | -iota) >> 31) & 1) ^ 1).astype(jnp.float32)
        ziv = jnp.zeros((LANES,), jnp.int32)

        def round_body(r, _0):
            wdw = wid * RN + r
            base = wdw * WIN
            pltpu.sync_copy(zeros_hbm.at[pl.ds(0, ACC)], acc)
            pltpu.sync_copy(zeros_hbm.at[pl.ds(0, ACCD)], accc)

            def sec_body(sec, _1):
                pltpu.sync_copy(tl_hbm.at[pl.ds(sec * S, S)], tbuf)
                pltpu.sync_copy(g_hbm.at[pl.ds(sec * S, S)], gbuf)

                def scan_body(i, carry):
                    cnt, stT, stG = carry
                    g0 = gbuf[pl.ds(i * 32, LANES)]
                    g1 = gbuf[pl.ds(i * 32 + LANES, LANES)]
                    gl0 = g0 - base
                    gl1 = g1 - base
                    neg0 = (gl0 | (WIN - 1 - gl0)) >> 31
                    neg1 = (gl1 | (WIN - 1 - gl1)) >> 31
                    bits = (onev & ~neg0) | (onevh & ~neg1)
                    for st_ in (8, 4, 2, 1):
                        bits = bits | bits[iota ^ st_]
                    bm0 = bits[0]
                    npop = _popcount32(bm0)
                    tv0 = tbuf[pl.ds(i * 32, LANES)]
                    tv1 = tbuf[pl.ds(i * 32 + LANES, LANES)]

                    def ext_body(j, c2):
                        bm, cnt2, sT, sG = c2
                        lane = _ctz(bm)
                        lsp = jnp.full((LANES,), lane & (LANES - 1))
                        him = jnp.full((LANES,), -(lane >> 4))
                        tval = (tv0[lsp] & ~him) | (tv1[lsp] & him)
                        gval = (gl0[lsp] & ~him) | (gl1[lsp] & him)
                        dslot = iota ^ (cnt2 & (LANES - 1))
                        keepm = (dslot | -dslot) >> 31
                        sT = (sT & keepm) | (tval & ~keepm)
                        sG = (sG & keepm) | (gval & ~keepm)

                        @pl.when((cnt2 & (LANES - 1)) == (LANES - 1))
                        def _():
                            tlist[pl.ds(cnt2 - (LANES - 1), LANES)] = sT
                            glist[pl.ds(cnt2 - (LANES - 1), LANES)] = sG

                        return (bm & (bm - 1), cnt2 + 1, sT, sG)

                    bm, cnt, stT, stG = lax.fori_loop(
                        0, npop, ext_body, (bm0, cnt, stT, stG))
                    return (cnt, stT, stG)

                cnt, stT, stG = lax.fori_loop(
                    0, NV // 2, scan_body, (jnp.int32(0), ziv, ziv))

                @pl.when((cnt & (LANES - 1)) != 0)
                def _():
                    fb = (cnt >> 4) << 4
                    tlist[pl.ds(fb, LANES)] = stT
                    glist[pl.ds(fb, LANES)] = stG

                cs = jnp.full((LANES,), cnt)
                nb = (cnt + K - 1) // K

                def drain_body(b, _2):
                    off = b * K
                    for t_ in range(K // LANES):
                        validm = ((iota + off + t_ * LANES) - cs) >> 31
                        tv = tlist[pl.ds(off + t_ * LANES, LANES)] & validm
                        tidx[pl.ds(t_ * LANES, LANES)] = tv
                    pltpu.async_copy(tab_hbm.at[tidx], trows, gsem).wait()

                    def edge_body(e, _3):
                        okf = jnp.full(
                            (LANES,),
                            ((((off + e) - cnt) >> 31) & 1).astype(jnp.float32))
                        off2 = off + ((e >> 4) << 4)
                        validm2 = ((iota + off2) - cs) >> 31
                        gvv = glist[pl.ds(off2, LANES)] & validm2
                        dsel = iota ^ (e & (LANES - 1))
                        sel = gvv & ~((dsel | -dsel) >> 31)
                        for st_ in (8, 4, 2, 1):
                            sel = sel | sel[iota ^ st_]
                        gl0 = sel[0]
                        abase = gl0 * W
                        for t in range(W // LANES):
                            ao = abase + t * LANES
                            acc[pl.ds(ao, LANES)] = (
                                acc[pl.ds(ao, LANES)] +
                                trows[e, pl.ds(t * LANES, LANES)] * okf)
                        cb = gl0 * LANES
                        accc[pl.ds(cb, LANES)] = (
                            accc[pl.ds(cb, LANES)] + okf * f0v)
                        return 0

                    lax.fori_loop(0, K, edge_body, 0)
                    return 0

                lax.fori_loop(0, nb, drain_body, 0)
                return 0

            lax.fori_loop(0, NSEC, sec_body, 0)
            pltpu.sync_copy(acc, sum_out.at[pl.ds(base * W, ACC)])
            pltpu.sync_copy(accc, cnt_out.at[pl.ds(base * LANES, ACCD)])
            return 0

        lax.fori_loop(0, RN, round_body, 0)

    zeros = jnp.zeros((ACC,), jnp.float32)
    sums, cnts = body(table, tl, g, zeros)
    return sums, cnts, SEGP


def kernel(body, face, r_hand, l_hand, ecg, flow, params, pose_batch_edge_index, pose_batch_vector, batch_edge_index, batch_edge_types):
    pf = params['pf']
    mf = params['mf']
    B, C = body.shape
    pose = jnp.stack([body, face, r_hand, l_hand], axis=1)
    n = B * 4
    x = pose.reshape(n, C)

    src, dst = pose_batch_edge_index[0], pose_batch_edge_index[1]
    w4 = jnp.concatenate([pf['tqW'], pf['tkW'], pf['tvW'], pf['tsW']], axis=1)
    b4 = jnp.concatenate([pf['tqb'], pf['tkb'], pf['tvb'], pf['tsb']])
    qkvs = _proj_tc(x, w4, b4)
    q = qkvs[:, :H * C]
    k = qkvs[:, H * C:2 * H * C]
    v = qkvs[:, 2 * H * C:3 * H * C]
    skip = qkvs[:, 3 * H * C:]
    numf, denf = _tconv_edge_sc(q, k, v, src, dst)
    num = numf.reshape(n, H * C)
    den = denf.reshape(n, LANES)[:, :H]
    out = num.reshape(n, H, C) / (den[:, :, None] + 1e-16)
    pfx = out.reshape(n, H * C) + skip
    pfx = jax.nn.relu(_ln(pfx, pf['n1g'], pf['n1b'])).reshape(B, 4, H * C)

    conf = jax.nn.sigmoid(jax.nn.relu(pose @ pf['cW1'] + pf['cb1']) @ pf['cW2'] + pf['cb2'])
    flat = (pfx * conf).reshape(B, -1)
    pooled = jax.nn.relu(flat @ pf['apW'] + pf['apb'])
    pooled = jax.nn.relu(_ln(pooled, pf['n2g'], pf['n2b']))
    fused = pooled @ pf['mlpW'] + pf['mlpb']
    cls = pooled @ pf['clsW'] + pf['clsb']

    xm = jnp.stack([ecg, flow, fused], axis=1)
    cp = mf['cma']
    qc = xm @ cp['Wq'] + cp['bq']
    kc = xm @ cp['Wk'] + cp['bk']
    vc = xm @ cp['Wv'] + cp['bv']
    attn = jax.nn.softmax(jnp.einsum('bnc,bmc->bnm', qc, kc) / np.sqrt(C), axis=-1)
    co = jnp.einsum('bnm,bmc->bnc', attn, vc)
    gate = jax.nn.sigmoid(jnp.concatenate([co, xm], axis=-1) @ cp['Wg'] + cp['bg'])
    vx = _ln(gate * co + (1.0 - gate) * xm, cp['ln_g'], cp['ln_b'])

    conf2 = jax.nn.sigmoid(jax.nn.relu(vx @ mf['cW1'] + mf['cb1']) @ mf['cW2'] + mf['cb2'])
    wx = vx * conf2
    x2 = jax.nn.relu(jnp.concatenate([xm, wx], axis=-1) @ mf['fmW'] + mf['fmb'])
    xn = _ln(x2, mf['nbg'], mf['nbb'])
    xnf = xn.reshape(-1, C)
    nm = xnf.shape[0]

    rg = mf['rgcn']
    rootp = xnf @ rg['root'] + rg['bias']
    t = jnp.einsum('nc,rcd->rnd', xnf, rg['W'])
    src2, dst2 = batch_edge_index[0], batch_edge_index[1]
    tl = batch_edge_types * nm + src2
    g = batch_edge_types * nm + dst2
    sums, cnts, segp = _rgcn_edge_sc(t.reshape(R * nm, C), tl, g, R * nm)
    s = sums.reshape(segp, C)[:R * nm].reshape(R, nm, C)
    cnt = cnts.reshape(segp, LANES)[:R * nm, :1].reshape(R, nm, 1)
    xr = rootp + (s / jnp.maximum(cnt, 1.0)).sum(0)
    xr = jax.nn.relu(_ln(xr.reshape(B, 3, C), mf['nag'], mf['nab']))

    den2 = jnp.maximum(conf2.sum(1), 1e-8)
    pooled2 = (xr * conf2).sum(1) / den2
    logits = pooled2 @ mf['headW'] + mf['headb']
    return cls, logits
